# baseline (speedup 1.0000x reference)
import functools

import jax
import jax.numpy as jnp
import numpy as np
from jax import lax
from jax.experimental import pallas as pl
from jax.experimental.pallas import tpu as pltpu
from jax.experimental.pallas import tpu_sc as plsc

D, FH, FW = 59, 28, 50
NX0, NX1, NX2 = 200, 400, 1
NCAM = 6
C = 80
PIX = FH * FW
PIXP = 1408
DROWS = 60
NROWS = NCAM * DROWS
NROWSP = 368
NSEG = NX0 * NX1 * NX2

ROWS_PER_CHUNK = 6
NCHUNK = NROWS // ROWS_PER_CHUNK
CHUNK_W = ROWS_PER_CHUNK * PIXP
CHUNKS_PER_CAM = DROWS // ROWS_PER_CHUNK

_DX = np.array([0.15, 0.15, 4.0], np.float32)
_BX = np.array([-15.0 + 0.075, -30.0 + 0.075, 0.0], np.float32)
_NXF = np.array([NX0, NX1, NX2], np.float32)
_LOWER = _BX - _DX / np.float32(2.0)
_UPPER = _LOWER + _DX * _NXF


def _stage1_body(dp_ref, u_ref, v_ref, ki_ref, rt_ref, rw_ref, flag_ref):
    def bf(x):
        return x.astype(jnp.bfloat16).astype(jnp.float32)

    n = pl.program_id(0)
    u = u_ref[0:1, :]
    v = v_ref[0:1, :]
    qx = (ki_ref[n, 0] * u + ki_ref[n, 1] * v) + ki_ref[n, 2]
    qy = (ki_ref[n, 3] * u + ki_ref[n, 4] * v) + ki_ref[n, 5]
    qz = (ki_ref[n, 6] * u + ki_ref[n, 7] * v) + ki_ref[n, 8]
    dval = lax.broadcasted_iota(jnp.int32, (DROWS, PIXP), 0).astype(
        jnp.float32) + 1.0
    cx = bf(qx * dval)
    cy = bf(qy * dval)
    cz = bf(qz * dval)
    ego = []
    for i in range(3):
        r0 = rt_ref[n, 4 * i + 0]
        r1 = rt_ref[n, 4 * i + 1]
        r2 = rt_ref[n, 4 * i + 2]
        t = rt_ref[n, 4 * i + 3]
        ego.append(((r0 * cx + r1 * cy) + r2 * cz) + t)
    kept = None
    for i in range(3):
        ki = (ego[i] >= _LOWER[i].item()) & (ego[i] < _UPPER[i].item())
        kept = ki if kept is None else (kept & ki)
    ix = jnp.clip((ego[0] - _LOWER[0].item()) / _DX[0].item(), 0.0, float(NX0 - 1))
    iy = jnp.clip((ego[1] - _LOWER[1].item()) / _DX[1].item(), 0.0, float(NX1 - 1))
    rank = ix.astype(jnp.int32) + iy.astype(jnp.int32) * NX0
    w = jnp.where(kept, dp_ref[0], 0.0)
    rank_f = lax.bitcast_convert_type(jnp.where(kept, rank, 0), jnp.float32)
    rw_ref[0] = jnp.stack([rank_f, w], axis=1)
    cnt = jnp.sum((w > 0.0).astype(jnp.int32), axis=1, keepdims=True)
    flag_ref[0] = jnp.broadcast_to(cnt, (DROWS, 128))


def _stage1(dp_pad, ugrid, vgrid, ki_pack, rt_pack):
    return pl.pallas_call(
        _stage1_body,
        grid=(NCAM,),
        in_specs=[
            pl.BlockSpec((1, DROWS, PIXP), lambda n: (n, 0, 0)),
            pl.BlockSpec((1, PIXP), lambda n: (0, 0)),
            pl.BlockSpec((1, PIXP), lambda n: (0, 0)),
            pl.BlockSpec(memory_space=pltpu.SMEM),
            pl.BlockSpec(memory_space=pltpu.SMEM),
        ],
        out_specs=[
            pl.BlockSpec((1, DROWS, 2, PIXP), lambda n: (n, 0, 0, 0)),
            pl.BlockSpec((1, DROWS, 128), lambda n: (n, 0, 0)),
        ],
        out_shape=[
            jax.ShapeDtypeStruct((NCAM, DROWS, 2, PIXP), jnp.float32),
            jax.ShapeDtypeStruct((NCAM, DROWS, 128), jnp.int32),
        ],
    )(dp_pad, ugrid, vgrid, ki_pack, rt_pack)


NBUF = 8
ZERO_THRESH = 24


def _stage2_body(rw_hbm, ftab_hbm, flags_hbm, out_hbm, acc, fbuf,
                 rwbuf, flags_vmem, rowlist_smem, sem_rw):
    nc = 2
    wid = lax.axis_index("s") * nc + lax.axis_index("c")

    pltpu.sync_copy(flags_hbm, flags_vmem)

    def scan_body(iv, cnt):
        v = flags_vmem[pl.ds(iv * 16, 16)]
        for k in range(16):
            i = iv * 16 + k
            f = v[k]

            @pl.when(f > 0)
            def _(i=i, cnt=cnt):
                rowlist_smem[cnt] = i

            cnt = cnt + (f > 0).astype(jnp.int32)
        return cnt

    cnt = lax.fori_loop(0, NROWSP // 16, scan_body, jnp.int32(0))

    def full_memset():
        @plsc.parallel_loop(0, NSEG, 16, unroll=8)
        def _(i):
            acc[pl.ds(i, 16)] = jnp.zeros((16,), jnp.float32)

    def start_row(j, b):
        row = rowlist_smem[j]
        off = pl.multiple_of(row * 2 * PIXP, 8)
        pltpu.make_async_copy(
            rw_hbm.at[pl.ds(off, 2 * PIXP)], rwbuf.at[b], sem_rw.at[b]).start()

    def wait_row(b):
        pltpu.make_async_copy(
            rw_hbm.at[pl.ds(0, 2 * PIXP)], rwbuf.at[b], sem_rw.at[b]).wait()

    def process_rows(zero_mode):
        for b in range(NBUF):
            @pl.when(b < cnt)
            def _(b=b):
                start_row(jnp.int32(b), b)

        ngroups = lax.div(cnt + (NBUF - 1), jnp.int32(NBUF))

        def group(g, _):
            for b in range(NBUF):
                j = g * NBUF + b

                @pl.when(j < cnt)
                def _(b=b, j=j):
                    wait_row(b)
                    row = rowlist_smem[j]
                    fbase = lax.div(row, jnp.int32(DROWS)) * PIXP

                    def vec_body(i, _):
                        for k in range(8):
                            o = (i * 8 + k) * 16
                            r = plsc.bitcast(rwbuf[b, pl.ds(o, 16)], jnp.int32)
                            if zero_mode:
                                plsc.store_scatter(
                                    acc, [r], jnp.zeros((16,), jnp.float32))
                            else:
                                w = rwbuf[b, pl.ds(PIXP + o, 16)]
                                f = fbuf[pl.ds(fbase + o, 16)]
                                plsc.addupdate_scatter(acc, [r], w * f)
                        return 0

                    lax.fori_loop(0, PIXP // 128, vec_body, 0)

                    @pl.when(j + NBUF < cnt)
                    def _(b=b, j=j):
                        start_row(j + NBUF, b)

            return 0

        lax.fori_loop(0, ngroups, group, 0)

    def run_channel(ch, first):
        foff = pl.multiple_of(ch * NCAM * PIXP, 8)
        pltpu.sync_copy(ftab_hbm.at[pl.ds(foff, NCAM * PIXP)], fbuf)
        if first:
            full_memset()
        else:
            @pl.when(cnt <= ZERO_THRESH)
            def _():
                process_rows(zero_mode=True)

            @pl.when(cnt > ZERO_THRESH)
            def _():
                full_memset()

        process_rows(zero_mode=False)
        ooff = pl.multiple_of(ch * NSEG, 8)
        pltpu.sync_copy(acc, out_hbm.at[pl.ds(ooff, NSEG)])

    run_channel(wid, True)
    run_channel(wid + 32, False)

    @pl.when(wid + 64 < C)
    def _():
        run_channel(wid + 64, False)


@functools.cache
def _make_stage2():
    return functools.partial(
        pl.kernel,
        out_type=jax.ShapeDtypeStruct((C * NSEG,), jnp.float32),
        mesh=plsc.VectorSubcoreMesh(core_axis_name="c", subcore_axis_name="s"),
        compiler_params=pltpu.CompilerParams(needs_layout_passes=False),
        scratch_types=[
            pltpu.VMEM((NSEG,), jnp.float32),
            pltpu.VMEM((NCAM * PIXP,), jnp.float32),
            pltpu.VMEM((NBUF, 2 * PIXP), jnp.float32),
            pltpu.VMEM((NROWSP,), jnp.int32),
            pltpu.SMEM((NROWSP,), jnp.int32),
            pltpu.SemaphoreType.DMA((NBUF,)),
        ],
    )(_stage2_body)


def kernel(img_feats, depth_probs, sensor2ego, intrinsics):
    B = img_feats.shape[0]
    k_inv = jnp.linalg.inv(intrinsics)

    xs = jnp.linspace(0.0, 799.0, FW)
    ys = jnp.linspace(0.0, 449.0, FH)
    ugrid = jnp.pad(jnp.broadcast_to(xs[None, :], (FH, FW)).reshape(1, PIX),
                    ((0, 0), (0, PIXP - PIX))).astype(jnp.float32)
    vgrid = jnp.pad(jnp.broadcast_to(ys[:, None], (FH, FW)).reshape(1, PIX),
                    ((0, 0), (0, PIXP - PIX))).astype(jnp.float32)
    ugrid = ugrid.astype(jnp.bfloat16).astype(jnp.float32)
    vgrid = vgrid.astype(jnp.bfloat16).astype(jnp.float32)

    outs = []
    for b in range(B):
        ki_pack = jnp.pad(k_inv[b].reshape(NCAM, 9), ((0, 0), (0, 7)))
        ki_pack = ki_pack.astype(jnp.bfloat16).astype(jnp.float32)
        rt_pack = jnp.pad(sensor2ego[b, :, :3, :4].reshape(NCAM, 12),
                          ((0, 0), (0, 4)))
        rt_pack = rt_pack.astype(jnp.bfloat16).astype(jnp.float32)
        dp_pad = jnp.pad(depth_probs[b].reshape(NCAM, D, PIX),
                         ((0, 0), (0, DROWS - D), (0, PIXP - PIX)))
        rw, flags = _stage1(dp_pad, ugrid, vgrid, ki_pack, rt_pack)

        ftab = jnp.pad(
            jnp.transpose(img_feats[b], (1, 0, 2, 3)).reshape(C, NCAM, PIX),
            ((0, 0), (0, 0), (0, PIXP - PIX)))
        flags1d = jnp.pad(flags[:, :, 0].reshape(-1), (0, NROWSP - NROWS))
        out = _make_stage2()(rw.reshape(-1), ftab.reshape(-1), flags1d)
        outs.append(out.reshape(C, NX2, NX1, NX0))
    return jnp.stack(outs)


# speedup vs baseline: 1.0462x; 1.0462x over previous
import functools

import jax
import jax.numpy as jnp
import numpy as np
from jax import lax
from jax.experimental import pallas as pl
from jax.experimental.pallas import tpu as pltpu
from jax.experimental.pallas import tpu_sc as plsc

D, FH, FW = 59, 28, 50
NX0, NX1, NX2 = 200, 400, 1
NCAM = 6
C = 80
PIX = FH * FW
PIXP = 1408
DROWS = 60
NROWS = NCAM * DROWS
NROWSP = 368
NSEG = NX0 * NX1 * NX2

ROWS_PER_CHUNK = 6
NCHUNK = NROWS // ROWS_PER_CHUNK
CHUNK_W = ROWS_PER_CHUNK * PIXP
CHUNKS_PER_CAM = DROWS // ROWS_PER_CHUNK

_DX = np.array([0.15, 0.15, 4.0], np.float32)
_BX = np.array([-15.0 + 0.075, -30.0 + 0.075, 0.0], np.float32)
_NXF = np.array([NX0, NX1, NX2], np.float32)
_LOWER = _BX - _DX / np.float32(2.0)
_UPPER = _LOWER + _DX * _NXF


def _stage1_body(dp_ref, u_ref, v_ref, ki_ref, rt_ref, rank_ref, w_ref,
                 flag_ref):
    def bf(x):
        return x.astype(jnp.bfloat16).astype(jnp.float32)

    n = pl.program_id(0)
    u = u_ref[0:1, :]
    v = v_ref[0:1, :]
    qx = (ki_ref[n, 0] * u + ki_ref[n, 1] * v) + ki_ref[n, 2]
    qy = (ki_ref[n, 3] * u + ki_ref[n, 4] * v) + ki_ref[n, 5]
    qz = (ki_ref[n, 6] * u + ki_ref[n, 7] * v) + ki_ref[n, 8]
    dval = lax.broadcasted_iota(jnp.int32, (DROWS, PIXP), 0).astype(
        jnp.float32) + 1.0
    cx = bf(qx * dval)
    cy = bf(qy * dval)
    cz = bf(qz * dval)
    ego = []
    for i in range(3):
        r0 = rt_ref[n, 4 * i + 0]
        r1 = rt_ref[n, 4 * i + 1]
        r2 = rt_ref[n, 4 * i + 2]
        t = rt_ref[n, 4 * i + 3]
        ego.append(((r0 * cx + r1 * cy) + r2 * cz) + t)
    kept = None
    for i in range(3):
        ki = (ego[i] >= _LOWER[i].item()) & (ego[i] < _UPPER[i].item())
        kept = ki if kept is None else (kept & ki)
    ix = jnp.clip((ego[0] - _LOWER[0].item()) / _DX[0].item(), 0.0, float(NX0 - 1))
    iy = jnp.clip((ego[1] - _LOWER[1].item()) / _DX[1].item(), 0.0, float(NX1 - 1))
    rank = ix.astype(jnp.int32) + iy.astype(jnp.int32) * NX0
    w = jnp.where(kept, dp_ref[0], 0.0)
    rank_ref[0] = jnp.where(kept, rank, 0)
    w_ref[0] = w
    cnt = jnp.sum((w > 0.0).astype(jnp.int32), axis=1, keepdims=True)
    flag_ref[0] = jnp.broadcast_to(cnt, (DROWS, 128))


def _stage1(dp_pad, ugrid, vgrid, ki_pack, rt_pack):
    return pl.pallas_call(
        _stage1_body,
        grid=(NCAM,),
        in_specs=[
            pl.BlockSpec((1, DROWS, PIXP), lambda n: (n, 0, 0)),
            pl.BlockSpec((1, PIXP), lambda n: (0, 0)),
            pl.BlockSpec((1, PIXP), lambda n: (0, 0)),
            pl.BlockSpec(memory_space=pltpu.SMEM),
            pl.BlockSpec(memory_space=pltpu.SMEM),
        ],
        out_specs=[
            pl.BlockSpec((1, DROWS, PIXP), lambda n: (n, 0, 0)),
            pl.BlockSpec((1, DROWS, PIXP), lambda n: (n, 0, 0)),
            pl.BlockSpec((1, DROWS, 128), lambda n: (n, 0, 0)),
        ],
        out_shape=[
            jax.ShapeDtypeStruct((NCAM, DROWS, PIXP), jnp.int32),
            jax.ShapeDtypeStruct((NCAM, DROWS, PIXP), jnp.float32),
            jax.ShapeDtypeStruct((NCAM, DROWS, 128), jnp.int32),
        ],
    )(dp_pad, ugrid, vgrid, ki_pack, rt_pack)


NBUF = 4
ZERO_THRESH = 24


def _stage2_body(rank_hbm, w_hbm, ftab_hbm, flags_hbm, out_hbm, acc, fbuf,
                 rbuf, wbuf, flags_vmem, rowlist_smem, sem_r, sem_w):
    nc = 2
    wid = lax.axis_index("s") * nc + lax.axis_index("c")

    pltpu.sync_copy(flags_hbm, flags_vmem)

    def scan_body(iv, cnt):
        v = flags_vmem[pl.ds(iv * 16, 16)]
        for k in range(16):
            i = iv * 16 + k
            f = v[k]

            @pl.when(f > 0)
            def _(i=i, cnt=cnt):
                rowlist_smem[cnt] = i

            cnt = cnt + (f > 0).astype(jnp.int32)
        return cnt

    cnt = lax.fori_loop(0, NROWSP // 16, scan_body, jnp.int32(0))

    def full_memset():
        @plsc.parallel_loop(0, NSEG, 16, unroll=8)
        def _(i):
            acc[pl.ds(i, 16)] = jnp.zeros((16,), jnp.float32)

    def start_row(j, b):
        row = rowlist_smem[j]
        off = pl.multiple_of(row * PIXP, 8)
        pltpu.make_async_copy(
            rank_hbm.at[pl.ds(off, PIXP)], rbuf.at[b], sem_r.at[b]).start()
        pltpu.make_async_copy(
            w_hbm.at[pl.ds(off, PIXP)], wbuf.at[b], sem_w.at[b]).start()

    def wait_row(b):
        pltpu.make_async_copy(
            rank_hbm.at[pl.ds(0, PIXP)], rbuf.at[b], sem_r.at[b]).wait()
        pltpu.make_async_copy(
            w_hbm.at[pl.ds(0, PIXP)], wbuf.at[b], sem_w.at[b]).wait()

    def process_rows(zero_mode):
        for b in range(NBUF):
            @pl.when(b < cnt)
            def _(b=b):
                start_row(jnp.int32(b), b)

        ngroups = lax.div(cnt + (NBUF - 1), jnp.int32(NBUF))

        def group(g, _):
            for b in range(NBUF):
                j = g * NBUF + b

                @pl.when(j < cnt)
                def _(b=b, j=j):
                    wait_row(b)
                    row = rowlist_smem[j]
                    fbase = lax.div(row, jnp.int32(DROWS)) * PIXP

                    def vec_body(i, _):
                        for k in range(8):
                            o = (i * 8 + k) * 16
                            r = rbuf[b, pl.ds(o, 16)]
                            if zero_mode:
                                plsc.store_scatter(
                                    acc, [r], jnp.zeros((16,), jnp.float32))
                            else:
                                w = wbuf[b, pl.ds(o, 16)]
                                f = fbuf[pl.ds(fbase + o, 16)]
                                plsc.addupdate_scatter(acc, [r], w * f)
                        return 0

                    lax.fori_loop(0, PIXP // 128, vec_body, 0)

                    @pl.when(j + NBUF < cnt)
                    def _(b=b, j=j):
                        start_row(j + NBUF, b)

            return 0

        lax.fori_loop(0, ngroups, group, 0)

    def run_channel(ch, first):
        foff = pl.multiple_of(ch * NCAM * PIXP, 8)
        pltpu.sync_copy(ftab_hbm.at[pl.ds(foff, NCAM * PIXP)], fbuf)
        if first:
            full_memset()
        else:
            @pl.when(cnt <= ZERO_THRESH)
            def _():
                process_rows(zero_mode=True)

            @pl.when(cnt > ZERO_THRESH)
            def _():
                full_memset()

        process_rows(zero_mode=False)
        ooff = pl.multiple_of(ch * NSEG, 8)
        pltpu.sync_copy(acc, out_hbm.at[pl.ds(ooff, NSEG)])

    run_channel(wid, True)
    run_channel(wid + 32, False)

    @pl.when(wid + 64 < C)
    def _():
        run_channel(wid + 64, False)


@functools.cache
def _make_stage2():
    return functools.partial(
        pl.kernel,
        out_type=jax.ShapeDtypeStruct((C * NSEG,), jnp.float32),
        mesh=plsc.VectorSubcoreMesh(core_axis_name="c", subcore_axis_name="s"),
        compiler_params=pltpu.CompilerParams(needs_layout_passes=False),
        scratch_types=[
            pltpu.VMEM((NSEG,), jnp.float32),
            pltpu.VMEM((NCAM * PIXP,), jnp.float32),
            pltpu.VMEM((NBUF, PIXP), jnp.int32),
            pltpu.VMEM((NBUF, PIXP), jnp.float32),
            pltpu.VMEM((NROWSP,), jnp.int32),
            pltpu.SMEM((NROWSP,), jnp.int32),
            pltpu.SemaphoreType.DMA((NBUF,)),
            pltpu.SemaphoreType.DMA((NBUF,)),
        ],
    )(_stage2_body)


def kernel(img_feats, depth_probs, sensor2ego, intrinsics):
    B = img_feats.shape[0]
    k_inv = jnp.linalg.inv(intrinsics)

    xs = jnp.linspace(0.0, 799.0, FW)
    ys = jnp.linspace(0.0, 449.0, FH)
    ugrid = jnp.pad(jnp.broadcast_to(xs[None, :], (FH, FW)).reshape(1, PIX),
                    ((0, 0), (0, PIXP - PIX))).astype(jnp.float32)
    vgrid = jnp.pad(jnp.broadcast_to(ys[:, None], (FH, FW)).reshape(1, PIX),
                    ((0, 0), (0, PIXP - PIX))).astype(jnp.float32)
    ugrid = ugrid.astype(jnp.bfloat16).astype(jnp.float32)
    vgrid = vgrid.astype(jnp.bfloat16).astype(jnp.float32)

    outs = []
    for b in range(B):
        ki_pack = jnp.pad(k_inv[b].reshape(NCAM, 9), ((0, 0), (0, 7)))
        ki_pack = ki_pack.astype(jnp.bfloat16).astype(jnp.float32)
        rt_pack = jnp.pad(sensor2ego[b, :, :3, :4].reshape(NCAM, 12),
                          ((0, 0), (0, 4)))
        rt_pack = rt_pack.astype(jnp.bfloat16).astype(jnp.float32)
        dp_pad = jnp.pad(depth_probs[b].reshape(NCAM, D, PIX),
                         ((0, 0), (0, DROWS - D), (0, PIXP - PIX)))
        ranks, weights, flags = _stage1(dp_pad, ugrid, vgrid, ki_pack, rt_pack)

        ftab = jnp.pad(
            jnp.transpose(img_feats[b], (1, 0, 2, 3)).reshape(C, NCAM, PIX),
            ((0, 0), (0, 0), (0, PIXP - PIX)))
        flags1d = jnp.pad(flags[:, :, 0].reshape(-1), (0, NROWSP - NROWS))
        out = _make_stage2()(ranks.reshape(-1), weights.reshape(-1),
                             ftab.reshape(-1), flags1d)
        outs.append(out.reshape(C, NX2, NX1, NX0))
    return jnp.stack(outs)
